# baseline (speedup 1.0000x reference)
import functools
import math

import jax
import jax.numpy as jnp
from jax import lax
from jax.experimental import pallas as pl
from jax.experimental.pallas import tpu as pltpu
from jax.experimental.pallas import tpu_sc as plsc

D_MODEL = 64
SCALE = math.sqrt(D_MODEL)

NC = 2
NS = 16
NW = NC * NS
LANES = 16

VOCAB = 1000000
PBLK = 2048
PHALF = PBLK // 2
NPB = -(-VOCAB // PBLK)
PROWS = NPB * PHALF

BATCH = 4096
SEQ = 200
IBLK = 128
NIB = BATCH // IBLK
Q = 4
SU_ROWS = Q * IBLK
NSUQ = NIB // Q
NSU_PER_W = SEQ * NSUQ // NW
HALF = SU_ROWS // 2
PITCH = D_MODEL + 1


def _tc_pack_body(t_ref, o_ref):
    ident = jnp.float32(
        lax.broadcasted_iota(jnp.int32, (D_MODEL, D_MODEL), 0)
        == lax.broadcasted_iota(jnp.int32, (D_MODEL, D_MODEL), 1)
    )
    tr = lax.dot_general(
        t_ref[...], ident, (((0,), (0,)), ((), ())),
        preferred_element_type=jnp.float32,
    )
    o_ref[...] = jnp.concatenate([tr[:PHALF], tr[PHALF:]], axis=1)


_tc_pack = pl.pallas_call(
    _tc_pack_body,
    grid=(NPB,),
    in_specs=[pl.BlockSpec((D_MODEL, PBLK), lambda i: (0, i))],
    out_specs=pl.BlockSpec((PHALF, 2 * D_MODEL), lambda i: (i, 0)),
    out_shape=jax.ShapeDtypeStruct((PROWS, 2 * D_MODEL), jnp.float32),
)


def _su_coords(su):
    s = su // NSUQ
    ib0 = (su % NSUQ) * Q
    return s, ib0


def _sc_body(x4_hbm, table_hbm, out_hbm, idx_v, rows_v, pad_v, tile_v,
             gsem, osem):
    wid = lax.axis_index("s") * NC + lax.axis_index("c")
    su0 = wid * NSU_PER_W
    lane = lax.iota(jnp.int32, LANES)

    def load_idx(su, b):
        s, ib0 = _su_coords(su)
        for k in range(Q):
            pltpu.sync_copy(
                x4_hbm.at[s // 8, ib0 + k, s % 8],
                idx_v.at[b, pl.ds(k * IBLK, IBLK)],
            )
        @pl.loop(0, SU_ROWS // LANES, unroll=4)
        def _remap(j):
            sl = pl.ds(j * LANES, LANES)
            v = idx_v[b, sl]
            idx_v[b, sl] = (
                (v & ~(PBLK - 1))
                + ((v & (PHALF - 1)) << 1)
                + ((v >> 10) & 1)
            )

    def gather(b):
        return pltpu.make_async_copy(
            table_hbm.at[idx_v.at[b]], rows_v.at[b], gsem.at[b]
        )

    def out_dma(su):
        s, ib0 = _su_coords(su)
        return pltpu.make_async_copy(
            tile_v, out_hbm.at[s, :, pl.ds(ib0, Q)], osem
        )

    def pass1(b, h):
        @pl.loop(0, HALF, unroll=4)
        def _row(r):
            vs = [
                rows_v[b, h * HALF + r, pl.ds(cq * LANES, LANES)]
                for cq in range(D_MODEL // LANES)
            ]
            for cq, v in enumerate(vs):
                pad_v[r, pl.ds(cq * LANES, LANES)] = v

    def pass2(h):
        @pl.loop(0, D_MODEL, unroll=2)
        def _col(c):
            cvec = jnp.full((LANES,), 0, jnp.int32) + c
            for jb4 in range(0, HALF // LANES, 8):
                vs = [
                    plsc.load_gather(
                        pad_v, [lane + ((jb4 + i) * LANES), cvec]
                    )
                    for i in range(8)
                ]
                for i, v in enumerate(vs):
                    jb = jb4 + i
                    tile_v[
                        c // 8,
                        h * 2 + jb // 8,
                        c % 8,
                        pl.ds((jb % 8) * LANES, LANES),
                    ] = v * SCALE

    for b in range(2):
        load_idx(su0 + b, b)
        gather(b).start()

    @pl.loop(0, NSU_PER_W, step=2)
    def _pipe(t):
        for b in range(2):
            su = su0 + t + b
            gather(b).wait()
            pass1(b, 0)

            @pl.when(t + b >= 1)
            def _():
                out_dma(su - 1).wait()

            pass2(0)
            pass1(b, 1)
            pass2(1)

            @pl.when(t + b + 2 < NSU_PER_W)
            def _():
                load_idx(su + 2, b)
                gather(b).start()

            out_dma(su).start()

    out_dma(su0 + NSU_PER_W - 1).wait()


_sc_gather = functools.partial(
    pl.kernel,
    out_type=jax.ShapeDtypeStruct(
        (SEQ, D_MODEL // 8, NIB, 8, IBLK), jnp.float32
    ),
    mesh=plsc.VectorSubcoreMesh(core_axis_name="c", subcore_axis_name="s"),
    scratch_types=[
        pltpu.VMEM((2, SU_ROWS), jnp.int32),
        pltpu.VMEM((2, SU_ROWS, D_MODEL), jnp.float32),
        pltpu.VMEM((HALF, PITCH), jnp.float32),
        pltpu.VMEM((D_MODEL // 8, Q, 8, IBLK), jnp.float32),
        pltpu.SemaphoreType.DMA((2,)),
        pltpu.SemaphoreType.DMA,
    ],
    compiler_params=pltpu.CompilerParams(
        use_tc_tiling_on_sc=False, needs_layout_passes=False
    ),
)(_sc_body)


def kernel(x, table):
    table2 = _tc_pack(table.T).reshape(2 * PROWS, D_MODEL)
    x4 = (
        x.T.astype(jnp.int32)
        .reshape(SEQ // 8, 8, NIB, IBLK)
        .transpose(0, 2, 1, 3)
    )
    out5 = _sc_gather(x4, table2)
    return out5.transpose(2, 4, 0, 1, 3).reshape(BATCH, SEQ, D_MODEL)


# speedup vs baseline: 1.0145x; 1.0145x over previous
import functools
import math

import jax
import jax.numpy as jnp
from jax import lax
from jax.experimental import pallas as pl
from jax.experimental.pallas import tpu as pltpu
from jax.experimental.pallas import tpu_sc as plsc

D_MODEL = 64
SCALE = math.sqrt(D_MODEL)

NC = 2
NS = 16
NW = NC * NS
LANES = 16

VOCAB = 1000000
PBLK = 2048
PHALF = PBLK // 2
NPB = -(-VOCAB // PBLK)
PROWS = NPB * PHALF

BATCH = 4096
SEQ = 200
IBLK = 128
NIB = BATCH // IBLK
Q = 4
SU_ROWS = Q * IBLK
NSUQ = NIB // Q
NSU_PER_W = SEQ * NSUQ // NW
HALF = SU_ROWS // 2
PITCH = D_MODEL + 1


def _tc_pack_body(t_ref, o_ref):
    tr = t_ref[...].T
    o_ref[...] = jnp.concatenate([tr[:PHALF], tr[PHALF:]], axis=1)


_tc_pack = pl.pallas_call(
    _tc_pack_body,
    grid=(NPB,),
    in_specs=[pl.BlockSpec((D_MODEL, PBLK), lambda i: (0, i))],
    out_specs=pl.BlockSpec((PHALF, 2 * D_MODEL), lambda i: (i, 0)),
    out_shape=jax.ShapeDtypeStruct((PROWS, 2 * D_MODEL), jnp.float32),
)


def _su_coords(su):
    s = su // NSUQ
    ib0 = (su % NSUQ) * Q
    return s, ib0


def _sc_body(x4_hbm, table_hbm, out_hbm, idx_v, rows_v, pad_v, tile_v,
             gsem, osem):
    wid = lax.axis_index("s") * NC + lax.axis_index("c")
    su0 = wid * NSU_PER_W
    lane = lax.iota(jnp.int32, LANES)

    def load_idx(su, b):
        s, ib0 = _su_coords(su)
        for k in range(Q):
            pltpu.sync_copy(
                x4_hbm.at[s // 8, ib0 + k, s % 8],
                idx_v.at[b, pl.ds(k * IBLK, IBLK)],
            )
        @pl.loop(0, SU_ROWS // LANES, unroll=4)
        def _remap(j):
            sl = pl.ds(j * LANES, LANES)
            v = idx_v[b, sl]
            idx_v[b, sl] = (
                (v & ~(PBLK - 1))
                + ((v & (PHALF - 1)) << 1)
                + ((v >> 10) & 1)
            )

    def gather(b):
        return pltpu.make_async_copy(
            table_hbm.at[idx_v.at[b]], rows_v.at[b], gsem.at[b]
        )

    def out_dma(su):
        s, ib0 = _su_coords(su)
        return pltpu.make_async_copy(
            tile_v, out_hbm.at[s, :, pl.ds(ib0, Q)], osem
        )

    def pass1(b, h):
        @pl.loop(0, HALF, unroll=4)
        def _row(r):
            vs = [
                rows_v[b, h * HALF + r, pl.ds(cq * LANES, LANES)]
                for cq in range(D_MODEL // LANES)
            ]
            for cq, v in enumerate(vs):
                pad_v[r, pl.ds(cq * LANES, LANES)] = v

    def pass2(h):
        @pl.loop(0, D_MODEL, unroll=2)
        def _col(c):
            cvec = jnp.full((LANES,), 0, jnp.int32) + c
            for jb4 in range(0, HALF // LANES, 8):
                vs = [
                    plsc.load_gather(
                        pad_v, [lane + ((jb4 + i) * LANES), cvec]
                    )
                    for i in range(8)
                ]
                for i, v in enumerate(vs):
                    jb = jb4 + i
                    tile_v[
                        c // 8,
                        h * 2 + jb // 8,
                        c % 8,
                        pl.ds((jb % 8) * LANES, LANES),
                    ] = v * SCALE

    for b in range(2):
        load_idx(su0 + b, b)
        gather(b).start()

    @pl.loop(0, NSU_PER_W, step=2)
    def _pipe(t):
        for b in range(2):
            su = su0 + t + b
            gather(b).wait()
            pass1(b, 0)

            @pl.when(t + b >= 1)
            def _():
                out_dma(su - 1).wait()

            pass2(0)
            pass1(b, 1)
            pass2(1)

            @pl.when(t + b + 2 < NSU_PER_W)
            def _():
                load_idx(su + 2, b)
                gather(b).start()

            out_dma(su).start()

    out_dma(su0 + NSU_PER_W - 1).wait()


_sc_gather = functools.partial(
    pl.kernel,
    out_type=jax.ShapeDtypeStruct(
        (SEQ, D_MODEL // 8, NIB, 8, IBLK), jnp.float32
    ),
    mesh=plsc.VectorSubcoreMesh(core_axis_name="c", subcore_axis_name="s"),
    scratch_types=[
        pltpu.VMEM((2, SU_ROWS), jnp.int32),
        pltpu.VMEM((2, SU_ROWS, D_MODEL), jnp.float32),
        pltpu.VMEM((HALF, PITCH), jnp.float32),
        pltpu.VMEM((D_MODEL // 8, Q, 8, IBLK), jnp.float32),
        pltpu.SemaphoreType.DMA((2,)),
        pltpu.SemaphoreType.DMA,
    ],
    compiler_params=pltpu.CompilerParams(
        use_tc_tiling_on_sc=False, needs_layout_passes=False
    ),
)(_sc_body)


def kernel(x, table):
    table2 = _tc_pack(table.T).reshape(2 * PROWS, D_MODEL)
    x4 = (
        x.T.astype(jnp.int32)
        .reshape(SEQ // 8, 8, NIB, IBLK)
        .transpose(0, 2, 1, 3)
    )
    out5 = _sc_gather(x4, table2)
    return out5.transpose(2, 4, 0, 1, 3).reshape(BATCH, SEQ, D_MODEL)


# speedup vs baseline: 1.3780x; 1.3583x over previous
import functools
import math

import jax
import jax.numpy as jnp
from jax import lax
from jax.experimental import pallas as pl
from jax.experimental.pallas import tpu as pltpu
from jax.experimental.pallas import tpu_sc as plsc

D_MODEL = 64
SCALE = math.sqrt(D_MODEL)

NC = 2
NS = 16
NW = NC * NS
LANES = 16

VOCAB = 1000000
PBLK = 8192
PHALF = PBLK // 2
NPB = -(-VOCAB // PBLK)
PROWS = NPB * PHALF

BATCH = 4096
SEQ = 200
IBLK = 128
NIB = BATCH // IBLK
Q = 4
SU_ROWS = Q * IBLK
NSUQ = NIB // Q
NSU_PER_W = SEQ * NSUQ // NW
HSHIFT = PHALF.bit_length() - 1
HALF = SU_ROWS // 2
PITCH = D_MODEL + 1


def _tc_pack_body(t_ref, o_ref):
    tr = t_ref[...].T
    o_ref[...] = jnp.concatenate([tr[:PHALF], tr[PHALF:]], axis=1)


_tc_pack = pl.pallas_call(
    _tc_pack_body,
    grid=(NPB,),
    in_specs=[pl.BlockSpec((D_MODEL, PBLK), lambda i: (0, i))],
    out_specs=pl.BlockSpec((PHALF, 2 * D_MODEL), lambda i: (i, 0)),
    out_shape=jax.ShapeDtypeStruct((PROWS, 2 * D_MODEL), jnp.float32),
)


def _su_coords(su):
    s = su // NSUQ
    ib0 = (su % NSUQ) * Q
    return s, ib0


def _sc_body(x4_hbm, table_hbm, out_hbm, idx_v, rows_v, pad_v, tile_v,
             gsem, osem, isem):
    wid = lax.axis_index("s") * NC + lax.axis_index("c")
    su0 = wid * NSU_PER_W
    lane = lax.iota(jnp.int32, LANES)

    def idx_dma(su, b, k):
        s, ib0 = _su_coords(su)
        return pltpu.make_async_copy(
            x4_hbm.at[s // 8, ib0 + k, s % 8],
            idx_v.at[b, pl.ds(k * IBLK, IBLK)],
            isem.at[b],
        )

    def load_idx(su, b):
        for k in range(Q):
            idx_dma(su, b, k).start()
        for k in range(Q):
            idx_dma(su, b, k).wait()
        @pl.loop(0, SU_ROWS // LANES, unroll=4)
        def _remap(j):
            sl = pl.ds(j * LANES, LANES)
            v = idx_v[b, sl]
            idx_v[b, sl] = (
                (v & ~(PBLK - 1))
                + ((v & (PHALF - 1)) << 1)
                + ((v >> HSHIFT) & 1)
            )

    def gather(b):
        return pltpu.make_async_copy(
            table_hbm.at[idx_v.at[b]], rows_v.at[b], gsem.at[b]
        )

    def out_dma(su):
        s, ib0 = _su_coords(su)
        return pltpu.make_async_copy(
            tile_v, out_hbm.at[s, :, pl.ds(ib0, Q)], osem
        )

    def pass1(b, h):
        @pl.loop(0, HALF, unroll=4)
        def _row(r):
            vs = [
                rows_v[b, h * HALF + r, pl.ds(cq * LANES, LANES)]
                for cq in range(D_MODEL // LANES)
            ]
            for cq, v in enumerate(vs):
                pad_v[r, pl.ds(cq * LANES, LANES)] = v

    def pass2(h):
        @pl.loop(0, D_MODEL, unroll=2)
        def _col(c):
            cvec = jnp.full((LANES,), 0, jnp.int32) + c
            for jb4 in range(0, HALF // LANES, 8):
                vs = [
                    plsc.load_gather(
                        pad_v, [lane + ((jb4 + i) * LANES), cvec]
                    )
                    for i in range(8)
                ]
                for i, v in enumerate(vs):
                    jb = jb4 + i
                    tile_v[
                        c // 8,
                        h * 2 + jb // 8,
                        c % 8,
                        pl.ds((jb % 8) * LANES, LANES),
                    ] = v * SCALE

    for b in range(2):
        load_idx(su0 + b, b)
        gather(b).start()

    @pl.loop(0, NSU_PER_W, step=2)
    def _pipe(t):
        for b in range(2):
            su = su0 + t + b
            gather(b).wait()
            pass1(b, 0)

            @pl.when(t + b >= 1)
            def _():
                out_dma(su - 1).wait()

            pass2(0)
            pass1(b, 1)
            pass2(1)

            @pl.when(t + b + 2 < NSU_PER_W)
            def _():
                load_idx(su + 2, b)
                gather(b).start()

            out_dma(su).start()

    out_dma(su0 + NSU_PER_W - 1).wait()


_sc_gather = functools.partial(
    pl.kernel,
    out_type=jax.ShapeDtypeStruct(
        (SEQ, D_MODEL // 8, NIB, 8, IBLK), jnp.float32
    ),
    mesh=plsc.VectorSubcoreMesh(core_axis_name="c", subcore_axis_name="s"),
    scratch_types=[
        pltpu.VMEM((2, SU_ROWS), jnp.int32),
        pltpu.VMEM((2, SU_ROWS, D_MODEL), jnp.float32),
        pltpu.VMEM((HALF, PITCH), jnp.float32),
        pltpu.VMEM((D_MODEL // 8, Q, 8, IBLK), jnp.float32),
        pltpu.SemaphoreType.DMA((2,)),
        pltpu.SemaphoreType.DMA,
        pltpu.SemaphoreType.DMA((2,)),
    ],
    compiler_params=pltpu.CompilerParams(
        use_tc_tiling_on_sc=False, needs_layout_passes=False
    ),
)(_sc_body)


def kernel(x, table):
    table2 = _tc_pack(table.T).reshape(2 * PROWS, D_MODEL)
    x4 = (
        x.T.astype(jnp.int32)
        .reshape(SEQ // 8, 8, NIB, IBLK)
        .transpose(0, 2, 1, 3)
    )
    out5 = _sc_gather(x4, table2)
    return out5.transpose(2, 4, 0, 1, 3).reshape(BATCH, SEQ, D_MODEL)


# speedup vs baseline: 1.4390x; 1.0443x over previous
import functools
import math

import jax
import jax.numpy as jnp
from jax import lax
from jax.experimental import pallas as pl
from jax.experimental.pallas import tpu as pltpu
from jax.experimental.pallas import tpu_sc as plsc

D_MODEL = 64
SCALE = math.sqrt(D_MODEL)

NC = 2
NS = 16
NW = NC * NS
LANES = 16

VOCAB = 1000000
PBLK = 16384
PHALF = PBLK // 2
NPB = -(-VOCAB // PBLK)
PROWS = NPB * PHALF

BATCH = 4096
SEQ = 200
IBLK = 128
NIB = BATCH // IBLK
Q = 4
SU_ROWS = Q * IBLK
NSUQ = NIB // Q
NSU_PER_W = SEQ * NSUQ // NW
HSHIFT = PHALF.bit_length() - 1
HALF = SU_ROWS // 2
PITCH = D_MODEL + 1


def _tc_pack_body(t_ref, o_ref):
    tr = t_ref[...].T
    o_ref[...] = jnp.concatenate([tr[:PHALF], tr[PHALF:]], axis=1)


_tc_pack = pl.pallas_call(
    _tc_pack_body,
    grid=(NPB,),
    in_specs=[pl.BlockSpec((D_MODEL, PBLK), lambda i: (0, i))],
    out_specs=pl.BlockSpec((PHALF, 2 * D_MODEL), lambda i: (i, 0)),
    out_shape=jax.ShapeDtypeStruct((PROWS, 2 * D_MODEL), jnp.float32),
)


def _su_coords(su):
    s = su // NSUQ
    ib0 = (su % NSUQ) * Q
    return s, ib0


def _sc_body(x4_hbm, table_hbm, out_hbm, idx_v, rows_v, pad_v, tile_v,
             gsem, osem, isem):
    wid = lax.axis_index("s") * NC + lax.axis_index("c")
    su0 = wid * NSU_PER_W
    lane = lax.iota(jnp.int32, LANES)

    def idx_dma(su, b, k):
        s, ib0 = _su_coords(su)
        return pltpu.make_async_copy(
            x4_hbm.at[s // 8, ib0 + k, s % 8],
            idx_v.at[b, pl.ds(k * IBLK, IBLK)],
            isem.at[b],
        )

    def load_idx(su, b):
        for k in range(Q):
            idx_dma(su, b, k).start()
        for k in range(Q):
            idx_dma(su, b, k).wait()
        @pl.loop(0, SU_ROWS // LANES, unroll=4)
        def _remap(j):
            sl = pl.ds(j * LANES, LANES)
            v = idx_v[b, sl]
            idx_v[b, sl] = (
                (v & ~(PBLK - 1))
                + ((v & (PHALF - 1)) << 1)
                + ((v >> HSHIFT) & 1)
            )

    def gather(b):
        return pltpu.make_async_copy(
            table_hbm.at[idx_v.at[b]], rows_v.at[b], gsem.at[b]
        )

    def out_dma(su):
        s, ib0 = _su_coords(su)
        return pltpu.make_async_copy(
            tile_v, out_hbm.at[s, :, pl.ds(ib0, Q)], osem
        )

    def pass1(b, h):
        @pl.loop(0, HALF, unroll=4)
        def _row(r):
            vs = [
                rows_v[b, h * HALF + r, pl.ds(cq * LANES, LANES)]
                for cq in range(D_MODEL // LANES)
            ]
            for cq, v in enumerate(vs):
                pad_v[r, pl.ds(cq * LANES, LANES)] = v

    def pass2(h):
        @pl.loop(0, D_MODEL, unroll=2)
        def _col(c):
            cvec = jnp.full((LANES,), 0, jnp.int32) + c
            for jb4 in range(0, HALF // LANES, 8):
                vs = [
                    plsc.load_gather(
                        pad_v, [lane + ((jb4 + i) * LANES), cvec]
                    )
                    for i in range(8)
                ]
                for i, v in enumerate(vs):
                    jb = jb4 + i
                    tile_v[
                        c // 8,
                        h * 2 + jb // 8,
                        c % 8,
                        pl.ds((jb % 8) * LANES, LANES),
                    ] = v * SCALE

    for b in range(2):
        load_idx(su0 + b, b)
        gather(b).start()

    @pl.loop(0, NSU_PER_W, step=2)
    def _pipe(t):
        for b in range(2):
            su = su0 + t + b
            gather(b).wait()
            pass1(b, 0)

            @pl.when(t + b >= 1)
            def _():
                out_dma(su - 1).wait()

            pass2(0)
            pass1(b, 1)
            pass2(1)

            @pl.when(t + b + 2 < NSU_PER_W)
            def _():
                load_idx(su + 2, b)
                gather(b).start()

            out_dma(su).start()

    out_dma(su0 + NSU_PER_W - 1).wait()


_sc_gather = functools.partial(
    pl.kernel,
    out_type=jax.ShapeDtypeStruct(
        (SEQ, D_MODEL // 8, NIB, 8, IBLK), jnp.float32
    ),
    mesh=plsc.VectorSubcoreMesh(core_axis_name="c", subcore_axis_name="s"),
    scratch_types=[
        pltpu.VMEM((2, SU_ROWS), jnp.int32),
        pltpu.VMEM((2, SU_ROWS, D_MODEL), jnp.float32),
        pltpu.VMEM((HALF, PITCH), jnp.float32),
        pltpu.VMEM((D_MODEL // 8, Q, 8, IBLK), jnp.float32),
        pltpu.SemaphoreType.DMA((2,)),
        pltpu.SemaphoreType.DMA,
        pltpu.SemaphoreType.DMA((2,)),
    ],
    compiler_params=pltpu.CompilerParams(
        use_tc_tiling_on_sc=False, needs_layout_passes=False
    ),
)(_sc_body)


def kernel(x, table):
    table2 = _tc_pack(table.T).reshape(2 * PROWS, D_MODEL)
    x4 = (
        x.T.astype(jnp.int32)
        .reshape(SEQ // 8, 8, NIB, IBLK)
        .transpose(0, 2, 1, 3)
    )
    out5 = _sc_gather(x4, table2)
    return out5.transpose(2, 4, 0, 1, 3).reshape(BATCH, SEQ, D_MODEL)


# speedup vs baseline: 1.4701x; 1.0216x over previous
import functools
import math

import jax
import jax.numpy as jnp
from jax import lax
from jax.experimental import pallas as pl
from jax.experimental.pallas import tpu as pltpu
from jax.experimental.pallas import tpu_sc as plsc

D_MODEL = 64
SCALE = math.sqrt(D_MODEL)

NC = 2
NS = 16
NW = NC * NS
LANES = 16

VOCAB = 1000000
PBLK = 32768
PHALF = PBLK // 2
NPB = -(-VOCAB // PBLK)
PROWS = NPB * PHALF

BATCH = 4096
SEQ = 200
IBLK = 128
NIB = BATCH // IBLK
Q = 4
SU_ROWS = Q * IBLK
NSUQ = NIB // Q
NSU_PER_W = SEQ * NSUQ // NW
HSHIFT = PHALF.bit_length() - 1
HALF = SU_ROWS // 2
PITCH = D_MODEL + 1


def _tc_pack_body(t_ref, o_ref):
    tr = t_ref[...].T
    o_ref[...] = jnp.concatenate([tr[:PHALF], tr[PHALF:]], axis=1)


_tc_pack = pl.pallas_call(
    _tc_pack_body,
    grid=(NPB,),
    in_specs=[pl.BlockSpec((D_MODEL, PBLK), lambda i: (0, i))],
    out_specs=pl.BlockSpec((PHALF, 2 * D_MODEL), lambda i: (i, 0)),
    out_shape=jax.ShapeDtypeStruct((PROWS, 2 * D_MODEL), jnp.float32),
)


def _su_coords(su):
    s = su // NSUQ
    ib0 = (su % NSUQ) * Q
    return s, ib0


def _sc_body(x4_hbm, table_hbm, out_hbm, idx_v, rows_v, pad_v, tile_v,
             gsem, osem, isem):
    wid = lax.axis_index("s") * NC + lax.axis_index("c")
    su0 = wid * NSU_PER_W
    lane = lax.iota(jnp.int32, LANES)

    def idx_dma(su, b, k):
        s, ib0 = _su_coords(su)
        return pltpu.make_async_copy(
            x4_hbm.at[s // 8, ib0 + k, s % 8],
            idx_v.at[b, pl.ds(k * IBLK, IBLK)],
            isem.at[b],
        )

    def load_idx(su, b):
        for k in range(Q):
            idx_dma(su, b, k).start()
        for k in range(Q):
            idx_dma(su, b, k).wait()
        @pl.loop(0, SU_ROWS // LANES, unroll=4)
        def _remap(j):
            sl = pl.ds(j * LANES, LANES)
            v = idx_v[b, sl]
            idx_v[b, sl] = (
                (v & ~(PBLK - 1))
                + ((v & (PHALF - 1)) << 1)
                + ((v >> HSHIFT) & 1)
            )

    def gather(b):
        return pltpu.make_async_copy(
            table_hbm.at[idx_v.at[b]], rows_v.at[b], gsem.at[b]
        )

    def out_dma(su):
        s, ib0 = _su_coords(su)
        return pltpu.make_async_copy(
            tile_v, out_hbm.at[s, :, pl.ds(ib0, Q)], osem
        )

    def pass1(b, h):
        @pl.loop(0, HALF, unroll=4)
        def _row(r):
            vs = [
                rows_v[b, h * HALF + r, pl.ds(cq * LANES, LANES)]
                for cq in range(D_MODEL // LANES)
            ]
            for cq, v in enumerate(vs):
                pad_v[r, pl.ds(cq * LANES, LANES)] = v

    def pass2(h):
        @pl.loop(0, D_MODEL, unroll=2)
        def _col(c):
            cvec = jnp.full((LANES,), 0, jnp.int32) + c
            for jb4 in range(0, HALF // LANES, 8):
                vs = [
                    plsc.load_gather(
                        pad_v, [lane + ((jb4 + i) * LANES), cvec]
                    )
                    for i in range(8)
                ]
                for i, v in enumerate(vs):
                    jb = jb4 + i
                    tile_v[
                        c // 8,
                        h * 2 + jb // 8,
                        c % 8,
                        pl.ds((jb % 8) * LANES, LANES),
                    ] = v * SCALE

    for b in range(2):
        load_idx(su0 + b, b)
        gather(b).start()

    @pl.loop(0, NSU_PER_W, step=2)
    def _pipe(t):
        for b in range(2):
            su = su0 + t + b
            gather(b).wait()
            pass1(b, 0)

            @pl.when(t + b >= 1)
            def _():
                out_dma(su - 1).wait()

            pass2(0)
            pass1(b, 1)
            pass2(1)

            @pl.when(t + b + 2 < NSU_PER_W)
            def _():
                load_idx(su + 2, b)
                gather(b).start()

            out_dma(su).start()

    out_dma(su0 + NSU_PER_W - 1).wait()


_sc_gather = functools.partial(
    pl.kernel,
    out_type=jax.ShapeDtypeStruct(
        (SEQ, D_MODEL // 8, NIB, 8, IBLK), jnp.float32
    ),
    mesh=plsc.VectorSubcoreMesh(core_axis_name="c", subcore_axis_name="s"),
    scratch_types=[
        pltpu.VMEM((2, SU_ROWS), jnp.int32),
        pltpu.VMEM((2, SU_ROWS, D_MODEL), jnp.float32),
        pltpu.VMEM((HALF, PITCH), jnp.float32),
        pltpu.VMEM((D_MODEL // 8, Q, 8, IBLK), jnp.float32),
        pltpu.SemaphoreType.DMA((2,)),
        pltpu.SemaphoreType.DMA,
        pltpu.SemaphoreType.DMA((2,)),
    ],
    compiler_params=pltpu.CompilerParams(
        use_tc_tiling_on_sc=False, needs_layout_passes=False
    ),
)(_sc_body)


def kernel(x, table):
    table2 = _tc_pack(table.T).reshape(2 * PROWS, D_MODEL)
    x4 = (
        x.T.astype(jnp.int32)
        .reshape(SEQ // 8, 8, NIB, IBLK)
        .transpose(0, 2, 1, 3)
    )
    out5 = _sc_gather(x4, table2)
    return out5.transpose(2, 4, 0, 1, 3).reshape(BATCH, SEQ, D_MODEL)
